# baseline (speedup 1.0000x reference)
import functools

import jax
import jax.numpy as jnp
from jax import lax
from jax.experimental import pallas as pl
from jax.experimental.pallas import tpu as pltpu
from jax.experimental.pallas import tpu_sc as plsc

N = 10000
E = 640000
HID = 128
OUT = 64
B = 64

NC = 2
NS = 16
KH = 128
K = 2 * KH
NP = 10240
RPT = NP // NS

_mesh = functools.partial(
    plsc.VectorSubcoreMesh, core_axis_name="c", subcore_axis_name="s")


def _pad_edges(n, multiple):
    return ((n + multiple - 1) // multiple) * multiple


PE_PRE = _pad_edges(E, NC * NS * K)
EN = E + N
PE = _pad_edges(EN, NC * NS * K)


def _pre_body(d1_hbm, eap_hbm, z8_hbm, out_hbm, d_v, e_v, acc):
    c = lax.axis_index("c")
    t = lax.axis_index("s")
    pltpu.sync_copy(z8_hbm.at[pl.ds(t * RPT, RPT)], acc.at[pl.ds(t * RPT, RPT)])
    plsc.subcore_barrier()
    epw = PE_PRE // (NC * NS)
    nwin = epw // K
    base_e = (c * NS + t) * epw

    def win(g, carry):
        e0 = base_e + g * K
        pltpu.sync_copy(eap_hbm.at[pl.ds(e0, K)], e_v)
        for j in range(2):
            pltpu.sync_copy(d1_hbm.at[pl.ds(e0 + j * KH, KH)], d_v.at[j])
            pltpu.sync_copy(e_v.at[pl.ds(j * KH, KH)],
                            acc.at[d_v.at[j]], add=True)
        return carry

    lax.fori_loop(0, nwin, win, 0)
    plsc.subcore_barrier()
    pltpu.sync_copy(acc.at[pl.ds(t * RPT, RPT)],
                    out_hbm.at[c, pl.ds(t * RPT, RPT)])


def _pre_pass(dst, ea):
    pad = PE_PRE - E
    d_pad = jnp.concatenate([dst, (jnp.arange(pad, dtype=jnp.int32) % N)])
    eap = jnp.concatenate(
        [ea, jnp.ones((E, 1), jnp.float32), jnp.zeros((E, 1), jnp.float32)],
        axis=1)
    eap = jnp.concatenate([eap, jnp.zeros((pad, 8), jnp.float32)], axis=0)
    z8 = jnp.zeros((NP, 8), jnp.float32)

    fn = pl.kernel(
        _pre_body,
        out_type=jax.ShapeDtypeStruct((NC, NP, 8), jnp.float32),
        mesh=_mesh(),
        scratch_types=[
            pltpu.VMEM((2, KH), jnp.int32),
            pltpu.VMEM((K, 8), jnp.float32),
            pltpu.VMEM_SHARED((NP, 8), jnp.float32),
        ],
    )
    return fn(d_pad, eap, z8)


def _make_edge_kernel(heads, feat, width):
    ch = feat // heads
    epw = PE // (NC * NS)
    nwin = epw // KH

    def body(s1_hbm, d1_hbm, q16_hbm, xp_hbm, adw_hbm, m16_hbm,
             zw_hbm, out_hbm,
             s_f, d_f, g_v, ad_v, q_v, m_v, acc, sem):
        c = lax.axis_index("c")
        t = lax.axis_index("s")
        pltpu.sync_copy(m16_hbm, m_v)
        pltpu.sync_copy(zw_hbm.at[pl.ds(t * RPT, RPT)],
                        acc.at[pl.ds(t * RPT, RPT)])
        plsc.subcore_barrier()
        mvec = m_v[...]
        base_e = (c * NS + t) * epw
        lanes = lax.iota(jnp.int32, 16)
        lane_ok = lanes < heads

        def win(g, carry):
            e0 = base_e + g * KH
            pltpu.sync_copy(s1_hbm.at[pl.ds(e0, KH)], s_f)
            pltpu.sync_copy(d1_hbm.at[pl.ds(e0, KH)], d_f)
            pltpu.sync_copy(q16_hbm.at[pl.ds(e0, KH)], q_v)
            pltpu.async_copy(xp_hbm.at[s_f], g_v, sem).wait()
            pltpu.async_copy(adw_hbm.at[d_f], ad_v, sem).wait()

            def ebody(e, cy):
                a1 = g_v[e, pl.ds(feat, 16)]
                al = a1 + ad_v[e] + q_v[e]
                al = jnp.where(al > 0, al, al * jnp.float32(0.2))
                pe = jnp.exp(al - mvec)
                ok = jnp.logical_and(lane_ok, e0 + e < EN)
                pe = jnp.where(ok, pe, jnp.float32(0.0))
                g_v[e, pl.ds(feat, 16)] = pe
                for h in range(heads):
                    pb = jnp.broadcast_to(pe[h], (16,))
                    for v2 in range(ch // 16):
                        col = h * ch + v2 * 16
                        g_v[e, pl.ds(col, 16)] = g_v[e, pl.ds(col, 16)] * pb
                return cy

            lax.fori_loop(0, KH, ebody, 0, unroll=4)
            pltpu.sync_copy(g_v, acc.at[d_f], add=True)
            return carry

        lax.fori_loop(0, nwin, win, 0)
        plsc.subcore_barrier()
        pltpu.sync_copy(acc.at[pl.ds(t * RPT, RPT)],
                        out_hbm.at[c, pl.ds(t * RPT, RPT)])

    return pl.kernel(
        body,
        out_type=jax.ShapeDtypeStruct((NC, NP, width), jnp.float32),
        mesh=_mesh(),
        compiler_params=pltpu.CompilerParams(
            needs_layout_passes=False, use_tc_tiling_on_sc=False),
        scratch_types=[
            pltpu.VMEM((KH,), jnp.int32),
            pltpu.VMEM((KH,), jnp.int32),
            pltpu.VMEM((KH, width), jnp.float32),
            pltpu.VMEM((KH, 16), jnp.float32),
            pltpu.VMEM((KH, 16), jnp.float32),
            pltpu.VMEM((16,), jnp.float32),
            pltpu.VMEM_SHARED((NP, width), jnp.float32),
            pltpu.SemaphoreType.DMA,
        ],
    )


_edge_kernel_12 = _make_edge_kernel(4, HID, 144)
_edge_kernel_3 = _make_edge_kernel(1, OUT, 80)


_NEG = -3.0e38


def _prep_tc(heads, feat, width, fin):
    blk = 1280
    grid = NP // blk

    def body(h_ref, w_ref, ast_ref, adt_ref, xp_ref, adw_ref, mx_ref):
        i = pl.program_id(0)
        ch = feat // heads
        xpb = jnp.dot(h_ref[...], w_ref[...],
                      preferred_element_type=jnp.float32)
        f_id = lax.broadcasted_iota(jnp.int32, (feat, heads), 0)
        h_id = lax.broadcasted_iota(jnp.int32, (feat, heads), 1)
        sel = (f_id // ch) == h_id
        amat_s = jnp.where(sel, jnp.tile(ast_ref[...], (heads, 1)), 0.0)
        amat_d = jnp.where(sel, jnp.tile(adt_ref[...], (heads, 1)), 0.0)
        asv = jnp.dot(xpb, amat_s, preferred_element_type=jnp.float32)
        adv = jnp.dot(xpb, amat_d, preferred_element_type=jnp.float32)
        xp_ref[:, :feat] = xpb
        xp_ref[:, feat:feat + heads] = asv
        xp_ref[:, feat + heads:] = jnp.zeros(
            (blk, width - feat - heads), jnp.float32)
        adw_ref[:, :heads] = adv
        adw_ref[:, heads:] = jnp.zeros((blk, 16 - heads), jnp.float32)
        mrow_s = jnp.pad(jnp.max(asv, axis=0, keepdims=True),
                         ((0, 0), (0, 128 - heads)), constant_values=_NEG)
        mrow_d = jnp.pad(jnp.max(adv, axis=0, keepdims=True),
                         ((0, 0), (0, 128 - heads)), constant_values=_NEG)
        cur = jnp.pad(jnp.concatenate([mrow_s, mrow_d], axis=0),
                      ((0, 6), (0, 0)), constant_values=_NEG)

        @pl.when(i == 0)
        def _():
            mx_ref[...] = cur

        @pl.when(i > 0)
        def _():
            mx_ref[...] = jnp.maximum(mx_ref[...], cur)

    return pl.pallas_call(
        body,
        grid=(grid,),
        in_specs=[
            pl.BlockSpec((blk, fin), lambda i: (i, 0)),
            pl.BlockSpec((fin, feat), lambda i: (0, 0)),
            pl.BlockSpec((feat // heads, heads), lambda i: (0, 0)),
            pl.BlockSpec((feat // heads, heads), lambda i: (0, 0)),
        ],
        out_specs=[
            pl.BlockSpec((blk, width), lambda i: (i, 0)),
            pl.BlockSpec((blk, 16), lambda i: (i, 0)),
            pl.BlockSpec((8, 128), lambda i: (0, 0)),
        ],
        out_shape=[
            jax.ShapeDtypeStruct((NP, width), jnp.float32),
            jax.ShapeDtypeStruct((NP, 16), jnp.float32),
            jax.ShapeDtypeStruct((8, 128), jnp.float32),
        ],
    )


_prep_k1 = _prep_tc(4, HID, 144, 25)
_prep_k2 = _prep_tc(4, HID, 144, HID)
_prep_k3 = _prep_tc(1, OUT, 80, HID)


def _q_tc(rows, from_pre):
    blk = 1280
    grid = rows // blk

    def body(e_ref, we1_ref, ae1_ref, we2_ref, ae2_ref, we3_ref, ae3_ref,
             q1_ref, q2_ref, q3_ref, mx_ref):
        i = pl.program_id(0)
        if from_pre:
            s = e_ref[0] + e_ref[1]
            deg = jnp.maximum(s[:, 6:7], 1.0)
            ea = s[:, :6] / deg
        else:
            ea = e_ref[:, :6]

        def fold(we_ref, ae_ref, heads, ch):
            w3 = we_ref[...]
            at = ae_ref[...]
            f_id = lax.broadcasted_iota(jnp.int32, (heads * ch, heads), 0)
            h_id = lax.broadcasted_iota(jnp.int32, (heads * ch, heads), 1)
            sel = (f_id // ch) == h_id
            amat = jnp.where(sel, jnp.tile(at, (heads, 1)), 0.0)
            return jnp.dot(w3, amat, preferred_element_type=jnp.float32)

        ae_mats = [fold(we1_ref, ae1_ref, 4, 32),
                   fold(we2_ref, ae2_ref, 4, 32),
                   fold(we3_ref, ae3_ref, 1, 64)]
        rows_m = []
        for (q_ref, am, heads) in ((q1_ref, ae_mats[0], 4),
                                   (q2_ref, ae_mats[1], 4),
                                   (q3_ref, ae_mats[2], 1)):
            q = jnp.dot(ea, am, preferred_element_type=jnp.float32)
            q_ref[:, :heads] = q
            q_ref[:, heads:] = jnp.zeros((blk, 16 - heads), jnp.float32)
            rows_m.append(jnp.pad(jnp.max(q, axis=0, keepdims=True),
                                  ((0, 0), (0, 128 - heads)),
                                  constant_values=_NEG))
        cur = jnp.pad(jnp.concatenate(rows_m, axis=0), ((0, 5), (0, 0)),
                      constant_values=_NEG)

        @pl.when(i == 0)
        def _():
            mx_ref[...] = cur

        @pl.when(i > 0)
        def _():
            mx_ref[...] = jnp.maximum(mx_ref[...], cur)

    e_spec = (pl.BlockSpec((2, blk, 8), lambda i: (0, i, 0)) if from_pre
              else pl.BlockSpec((blk, 8), lambda i: (i, 0)))
    return pl.pallas_call(
        body,
        grid=(grid,),
        in_specs=[
            e_spec,
            pl.BlockSpec((6, HID), lambda i: (0, 0)),
            pl.BlockSpec((32, 4), lambda i: (0, 0)),
            pl.BlockSpec((6, HID), lambda i: (0, 0)),
            pl.BlockSpec((32, 4), lambda i: (0, 0)),
            pl.BlockSpec((6, OUT), lambda i: (0, 0)),
            pl.BlockSpec((64, 1), lambda i: (0, 0)),
        ],
        out_specs=[
            pl.BlockSpec((blk, 16), lambda i: (i, 0)),
            pl.BlockSpec((blk, 16), lambda i: (i, 0)),
            pl.BlockSpec((blk, 16), lambda i: (i, 0)),
            pl.BlockSpec((8, 128), lambda i: (0, 0)),
        ],
        out_shape=[
            jax.ShapeDtypeStruct((rows, 16), jnp.float32),
            jax.ShapeDtypeStruct((rows, 16), jnp.float32),
            jax.ShapeDtypeStruct((rows, 16), jnp.float32),
            jax.ShapeDtypeStruct((8, 128), jnp.float32),
        ],
    )


def _fin_tc(heads, feat, width, with_bn):
    blk = 1000
    grid = N // blk

    def body(o_ref, b_ref, g_ref, be_ref, h_ref):
        ch = feat // heads
        acc = o_ref[0] + o_ref[1]
        den = acc[:, feat:feat + heads] + jnp.float32(1e-16)
        f_id = lax.broadcasted_iota(jnp.int32, (heads, feat), 1)
        h_id = lax.broadcasted_iota(jnp.int32, (heads, feat), 0)
        expand = jnp.where((f_id // ch) == h_id, 1.0, 0.0)
        den_e = jnp.dot(den, expand, preferred_element_type=jnp.float32)
        h = acc[:, :feat] / den_e + b_ref[...]
        if with_bn:
            h = h / jnp.sqrt(jnp.float32(1.0 + 1e-5)) * g_ref[...] + be_ref[...]
        h_ref[...] = jnp.where(h > 0, h, jnp.exp(h) - 1.0)

    return pl.pallas_call(
        body,
        grid=(grid,),
        in_specs=[
            pl.BlockSpec((2, blk, width), lambda i: (0, i, 0)),
            pl.BlockSpec((1, feat), lambda i: (0, 0)),
            pl.BlockSpec((1, feat), lambda i: (0, 0)),
            pl.BlockSpec((1, feat), lambda i: (0, 0)),
        ],
        out_specs=pl.BlockSpec((blk, feat), lambda i: (i, 0)),
        out_shape=jax.ShapeDtypeStruct((N, feat), jnp.float32),
    )


_fin_k1 = _fin_tc(4, HID, 144, True)
_fin_k3 = _fin_tc(1, OUT, 80, False)


def _pool_tc():
    blk = 1000
    grid = N // blk

    def body(h_ref, rum_ref, bat_ref, o_ref):
        i = pl.program_id(0)

        @pl.when(i == 0)
        def _():
            o_ref[...] = jnp.full((B, OUT), -jnp.inf, jnp.float32)

        hb = h_ref[...]
        seg = jnp.where(rum_ref[...] == 1, bat_ref[...], B)
        rid = lax.broadcasted_iota(jnp.int32, (B, 1), 0)
        acc = o_ref[...]
        for b in range(B):
            mb = jnp.max(jnp.where(seg == b, hb, -jnp.inf), axis=0,
                         keepdims=True)
            acc = jnp.where(rid == b, jnp.maximum(acc, mb), acc)
        o_ref[...] = acc

        @pl.when(i == grid - 1)
        def _():
            v = o_ref[...]
            o_ref[...] = jnp.where(jnp.isneginf(v), 0.0, v)

    return pl.pallas_call(
        body,
        grid=(grid,),
        in_specs=[
            pl.BlockSpec((blk, OUT), lambda i: (i, 0)),
            pl.BlockSpec((blk, 1), lambda i: (i, 0)),
            pl.BlockSpec((blk, 1), lambda i: (i, 0)),
        ],
        out_specs=pl.BlockSpec((B, OUT), lambda i: (0, 0)),
        out_shape=jax.ShapeDtypeStruct((B, OUT), jnp.float32),
    )


_pool_k = _pool_tc()

_Z144 = None


def _gat_layer(h_pad, s1, d1, q16, qmax, W, ast, adt, bias, g, be, heads,
               prep_fn, fin_fn):
    feat = W.shape[1]
    width = 144 if heads == 4 else 80
    xp_pad, adw, mx = prep_fn(h_pad, W, ast, adt)
    m = mx[0, :heads] + mx[1, :heads] + qmax
    m16 = jnp.zeros((16,), jnp.float32).at[:heads].set(m)
    zw = jnp.zeros((NP, width), jnp.float32)

    fn = _edge_kernel_12 if heads == 4 else _edge_kernel_3
    o = fn(s1, d1, q16, xp_pad, adw, m16, zw)
    return fin_fn(o, bias.reshape(1, feat), g.reshape(1, feat),
                  be.reshape(1, feat))


def kernel(x, edge_index, edge_attr, repeat_unit_mask, batch,
           W1, as1, ad1, We1, ae1, b1, g1, be1,
           W2, as2, ad2, We2, ae2, b2, g2, be2,
           W3, as3, ad3, We3, ae3, b3):
    src, dst = edge_index[0], edge_index[1]
    pre = _pre_pass(dst, edge_attr)

    sl = jnp.arange(N, dtype=jnp.int32)
    pad = PE - EN
    s1 = jnp.concatenate([src, sl, jnp.zeros((pad,), jnp.int32)])
    d1 = jnp.concatenate([dst, sl, (jnp.arange(pad, dtype=jnp.int32) % N)])

    eap8 = jnp.concatenate(
        [edge_attr, jnp.ones((E, 1), jnp.float32),
         jnp.zeros((E, 1), jnp.float32)], axis=1)
    EPAD = _pad_edges(E, 1280)
    eap8 = jnp.concatenate(
        [eap8, jnp.zeros((EPAD - E, 8), jnp.float32)], axis=0)
    qe1, qe2, qe3, mxe = _q_edges(eap8, We1, ae1.T, We2, ae2.T, We3, ae3.T)
    ql1, ql2, ql3, mxl = _q_loops(pre, We1, ae1.T, We2, ae2.T, We3, ae3.T)
    zq = jnp.zeros((PE - E - NP, 16), jnp.float32)
    q16_1 = jnp.concatenate([qe1[:E], ql1, zq], axis=0)
    q16_2 = jnp.concatenate([qe2[:E], ql2, zq], axis=0)
    q16_3 = jnp.concatenate([qe3[:E], ql3, zq], axis=0)
    qmax = jnp.maximum(mxe, mxl)
    qm1, qm2, qm3 = qmax[0, :4], qmax[1, :4], qmax[2, :1]

    xpad = jnp.concatenate([x, jnp.zeros((NP - N, 25), jnp.float32)], axis=0)
    h = _gat_layer(xpad, s1, d1, q16_1, qm1, W1, as1.T, ad1.T, b1, g1, be1,
                   4, _prep_k1, _fin_k1)
    hpad = jnp.concatenate([h, jnp.zeros((NP - N, HID), jnp.float32)], axis=0)
    h = _gat_layer(hpad, s1, d1, q16_2, qm2, W2, as2.T, ad2.T, b2, g2, be2,
                   4, _prep_k2, _fin_k1)
    hpad = jnp.concatenate([h, jnp.zeros((NP - N, HID), jnp.float32)], axis=0)
    h = _gat_layer(hpad, s1, d1, q16_3, qm3, W3, as3.T, ad3.T, b3, b3, b3,
                   1, _prep_k3, _fin_k3)

    out = _pool_k(h, repeat_unit_mask.reshape(N, 1), batch.reshape(N, 1))
    return out


_q_edges = _q_tc(_pad_edges(E, 1280), False)
_q_loops = _q_tc(NP, True)


# speedup vs baseline: 1.3214x; 1.3214x over previous
import functools

import jax
import jax.numpy as jnp
from jax import lax
from jax.experimental import pallas as pl
from jax.experimental.pallas import tpu as pltpu
from jax.experimental.pallas import tpu_sc as plsc

N = 10000
E = 640000
HID = 128
OUT = 64
B = 64

NC = 2
NS = 16
KH = 128
K = 2 * KH
KE = 96
NP = 10240
RPT = NP // NS

_mesh = functools.partial(
    plsc.VectorSubcoreMesh, core_axis_name="c", subcore_axis_name="s")


def _pad_edges(n, multiple):
    return ((n + multiple - 1) // multiple) * multiple


PE_PRE = _pad_edges(E, NC * NS * K)
EN = E + N
PE = _pad_edges(EN, NC * NS * KE)


def _pre_body(d1_hbm, eap_hbm, z8_hbm, out_hbm, d_v, e_v, acc):
    c = lax.axis_index("c")
    t = lax.axis_index("s")
    pltpu.sync_copy(z8_hbm.at[pl.ds(t * RPT, RPT)], acc.at[pl.ds(t * RPT, RPT)])
    plsc.subcore_barrier()
    epw = PE_PRE // (NC * NS)
    nwin = epw // K
    base_e = (c * NS + t) * epw

    def win(g, carry):
        e0 = base_e + g * K
        pltpu.sync_copy(eap_hbm.at[pl.ds(e0, K)], e_v)
        for j in range(2):
            pltpu.sync_copy(d1_hbm.at[pl.ds(e0 + j * KH, KH)], d_v.at[j])
            pltpu.sync_copy(e_v.at[pl.ds(j * KH, KH)],
                            acc.at[d_v.at[j]], add=True)
        return carry

    lax.fori_loop(0, nwin, win, 0)
    plsc.subcore_barrier()
    pltpu.sync_copy(acc.at[pl.ds(t * RPT, RPT)],
                    out_hbm.at[c, pl.ds(t * RPT, RPT)])


def _pre_pass(dst, ea):
    pad = PE_PRE - E
    d_pad = jnp.concatenate([dst, (jnp.arange(pad, dtype=jnp.int32) % N)])
    eap = jnp.concatenate(
        [ea, jnp.ones((E, 1), jnp.float32), jnp.zeros((E, 1), jnp.float32)],
        axis=1)
    eap = jnp.concatenate([eap, jnp.zeros((pad, 8), jnp.float32)], axis=0)
    z8 = jnp.zeros((NP, 8), jnp.float32)

    fn = pl.kernel(
        _pre_body,
        out_type=jax.ShapeDtypeStruct((NC, NP, 8), jnp.float32),
        mesh=_mesh(),
        scratch_types=[
            pltpu.VMEM((2, KH), jnp.int32),
            pltpu.VMEM((K, 8), jnp.float32),
            pltpu.VMEM_SHARED((NP, 8), jnp.float32),
        ],
    )
    return fn(d_pad, eap, z8)


def _make_edge_kernel(heads, feat, width):
    ch = feat // heads
    epw = PE // (NC * NS)
    nwin = epw // KE
    assert nwin % 2 == 0

    def body(s1_hbm, d1_hbm, q16_hbm, xp_hbm, adw_hbm, m16_hbm,
             zw_hbm, out_hbm,
             s_f, d_f, g_v, ad_v, q_v, m_v, acc, sg0, sg1, ss0, ss1):
        sem_g = (sg0, sg1)
        sem_s = (ss0, ss1)
        c = lax.axis_index("c")
        t = lax.axis_index("s")
        pltpu.sync_copy(m16_hbm, m_v)
        pltpu.sync_copy(zw_hbm.at[pl.ds(t * RPT, RPT)],
                        acc.at[pl.ds(t * RPT, RPT)])
        plsc.subcore_barrier()
        mvec = m_v[...]
        base_e = (c * NS + t) * epw
        lanes = lax.iota(jnp.int32, 16)
        lane_ok = lanes < heads

        def loads_sq(w, b):
            pltpu.sync_copy(s1_hbm.at[pl.ds(base_e + w * KE, KE)], s_f.at[b])
            pltpu.sync_copy(q16_hbm.at[pl.ds(base_e + w * KE, KE)], q_v.at[b])

        def load_d(w, b):
            pltpu.sync_copy(d1_hbm.at[pl.ds(base_e + w * KE, KE)], d_f.at[b])

        def issue_gathers(b):
            pltpu.async_copy(xp_hbm.at[s_f.at[b]], g_v.at[b], sem_g[b])
            pltpu.async_copy(adw_hbm.at[d_f.at[b]], ad_v.at[b], sem_g[b])

        def drain_gathers(b):
            pltpu.make_async_copy(
                xp_hbm.at[s_f.at[b]], g_v.at[b], sem_g[b]).wait()
            pltpu.make_async_copy(
                adw_hbm.at[d_f.at[b]], ad_v.at[b], sem_g[b]).wait()

        def issue_scatter(b):
            pltpu.async_copy(g_v.at[b], acc.at[d_f.at[b]], sem_s[b], add=True)

        def drain_scatter(b):
            pltpu.make_async_copy(
                g_v.at[b], acc.at[d_f.at[b]], sem_s[b]).wait()

        def compute(w, b):
            e0 = base_e + w * KE

            def ebody(e, cy):
                a1 = g_v[b, e, pl.ds(feat, 16)]
                al = a1 + ad_v[b, e] + q_v[b, e]
                al = jnp.where(al > 0, al, al * jnp.float32(0.2))
                pe = jnp.exp(al - mvec)
                ok = jnp.logical_and(lane_ok, e0 + e < EN)
                pe = jnp.where(ok, pe, jnp.float32(0.0))
                g_v[b, e, pl.ds(feat, 16)] = pe
                for h in range(heads):
                    pb = jnp.broadcast_to(pe[h], (16,))
                    for v2 in range(ch // 16):
                        col = h * ch + v2 * 16
                        g_v[b, e, pl.ds(col, 16)] = (
                            g_v[b, e, pl.ds(col, 16)] * pb)
                return cy

            lax.fori_loop(0, KE, ebody, 0, unroll=4)

        loads_sq(0, 0)
        load_d(0, 0)
        issue_gathers(0)
        loads_sq(1, 1)

        def win2(g2, carry):
            for b in (0, 1):
                w = g2 * 2 + b
                b2 = 1 - b
                drain_gathers(b)

                @pl.when(w > 0)
                def _():
                    drain_scatter(b2)

                @pl.when(w + 1 < nwin)
                def _():
                    load_d(w + 1, b2)
                    issue_gathers(b2)

                compute(w, b)
                issue_scatter(b)

                @pl.when(w + 2 < nwin)
                def _():
                    loads_sq(w + 2, b)
            return carry

        lax.fori_loop(0, nwin // 2, win2, 0)
        drain_scatter(1)
        plsc.subcore_barrier()
        pltpu.sync_copy(acc.at[pl.ds(t * RPT, RPT)],
                        out_hbm.at[c, pl.ds(t * RPT, RPT)])

    return pl.kernel(
        body,
        out_type=jax.ShapeDtypeStruct((NC, NP, width), jnp.float32),
        mesh=_mesh(),
        compiler_params=pltpu.CompilerParams(
            needs_layout_passes=False, use_tc_tiling_on_sc=False),
        scratch_types=[
            pltpu.VMEM((2, KE), jnp.int32),
            pltpu.VMEM((2, KE), jnp.int32),
            pltpu.VMEM((2, KE, width), jnp.float32),
            pltpu.VMEM((2, KE, 16), jnp.float32),
            pltpu.VMEM((2, KE, 16), jnp.float32),
            pltpu.VMEM((16,), jnp.float32),
            pltpu.VMEM_SHARED((NP, width), jnp.float32),
            pltpu.SemaphoreType.DMA,
            pltpu.SemaphoreType.DMA,
            pltpu.SemaphoreType.DMA,
            pltpu.SemaphoreType.DMA,
        ],
    )


_edge_kernel_12 = _make_edge_kernel(4, HID, 144)
_edge_kernel_3 = _make_edge_kernel(1, OUT, 80)


_NEG = -3.0e38


def _prep_tc(heads, feat, width, fin):
    blk = 1280
    grid = NP // blk

    def body(h_ref, w_ref, ast_ref, adt_ref, xp_ref, adw_ref, mx_ref):
        i = pl.program_id(0)
        ch = feat // heads
        xpb = jnp.dot(h_ref[...], w_ref[...],
                      preferred_element_type=jnp.float32)
        f_id = lax.broadcasted_iota(jnp.int32, (feat, heads), 0)
        h_id = lax.broadcasted_iota(jnp.int32, (feat, heads), 1)
        sel = (f_id // ch) == h_id
        amat_s = jnp.where(sel, jnp.tile(ast_ref[...], (heads, 1)), 0.0)
        amat_d = jnp.where(sel, jnp.tile(adt_ref[...], (heads, 1)), 0.0)
        asv = jnp.dot(xpb, amat_s, preferred_element_type=jnp.float32)
        adv = jnp.dot(xpb, amat_d, preferred_element_type=jnp.float32)
        xp_ref[:, :feat] = xpb
        xp_ref[:, feat:feat + heads] = asv
        xp_ref[:, feat + heads:] = jnp.zeros(
            (blk, width - feat - heads), jnp.float32)
        adw_ref[:, :heads] = adv
        adw_ref[:, heads:] = jnp.zeros((blk, 16 - heads), jnp.float32)
        mrow_s = jnp.pad(jnp.max(asv, axis=0, keepdims=True),
                         ((0, 0), (0, 128 - heads)), constant_values=_NEG)
        mrow_d = jnp.pad(jnp.max(adv, axis=0, keepdims=True),
                         ((0, 0), (0, 128 - heads)), constant_values=_NEG)
        cur = jnp.pad(jnp.concatenate([mrow_s, mrow_d], axis=0),
                      ((0, 6), (0, 0)), constant_values=_NEG)

        @pl.when(i == 0)
        def _():
            mx_ref[...] = cur

        @pl.when(i > 0)
        def _():
            mx_ref[...] = jnp.maximum(mx_ref[...], cur)

    return pl.pallas_call(
        body,
        grid=(grid,),
        in_specs=[
            pl.BlockSpec((blk, fin), lambda i: (i, 0)),
            pl.BlockSpec((fin, feat), lambda i: (0, 0)),
            pl.BlockSpec((feat // heads, heads), lambda i: (0, 0)),
            pl.BlockSpec((feat // heads, heads), lambda i: (0, 0)),
        ],
        out_specs=[
            pl.BlockSpec((blk, width), lambda i: (i, 0)),
            pl.BlockSpec((blk, 16), lambda i: (i, 0)),
            pl.BlockSpec((8, 128), lambda i: (0, 0)),
        ],
        out_shape=[
            jax.ShapeDtypeStruct((NP, width), jnp.float32),
            jax.ShapeDtypeStruct((NP, 16), jnp.float32),
            jax.ShapeDtypeStruct((8, 128), jnp.float32),
        ],
    )


_prep_k1 = _prep_tc(4, HID, 144, 25)
_prep_k2 = _prep_tc(4, HID, 144, HID)
_prep_k3 = _prep_tc(1, OUT, 80, HID)


def _q_tc(rows, from_pre):
    blk = 1280
    grid = rows // blk

    def body(e_ref, we1_ref, ae1_ref, we2_ref, ae2_ref, we3_ref, ae3_ref,
             q1_ref, q2_ref, q3_ref, mx_ref):
        i = pl.program_id(0)
        if from_pre:
            s = e_ref[0] + e_ref[1]
            deg = jnp.maximum(s[:, 6:7], 1.0)
            ea = s[:, :6] / deg
        else:
            ea = e_ref[:, :6]

        def fold(we_ref, ae_ref, heads, ch):
            w3 = we_ref[...]
            at = ae_ref[...]
            f_id = lax.broadcasted_iota(jnp.int32, (heads * ch, heads), 0)
            h_id = lax.broadcasted_iota(jnp.int32, (heads * ch, heads), 1)
            sel = (f_id // ch) == h_id
            amat = jnp.where(sel, jnp.tile(at, (heads, 1)), 0.0)
            return jnp.dot(w3, amat, preferred_element_type=jnp.float32)

        ae_mats = [fold(we1_ref, ae1_ref, 4, 32),
                   fold(we2_ref, ae2_ref, 4, 32),
                   fold(we3_ref, ae3_ref, 1, 64)]
        rows_m = []
        for (q_ref, am, heads) in ((q1_ref, ae_mats[0], 4),
                                   (q2_ref, ae_mats[1], 4),
                                   (q3_ref, ae_mats[2], 1)):
            q = jnp.dot(ea, am, preferred_element_type=jnp.float32)
            q_ref[:, :heads] = q
            q_ref[:, heads:] = jnp.zeros((blk, 16 - heads), jnp.float32)
            rows_m.append(jnp.pad(jnp.max(q, axis=0, keepdims=True),
                                  ((0, 0), (0, 128 - heads)),
                                  constant_values=_NEG))
        cur = jnp.pad(jnp.concatenate(rows_m, axis=0), ((0, 5), (0, 0)),
                      constant_values=_NEG)

        @pl.when(i == 0)
        def _():
            mx_ref[...] = cur

        @pl.when(i > 0)
        def _():
            mx_ref[...] = jnp.maximum(mx_ref[...], cur)

    e_spec = (pl.BlockSpec((2, blk, 8), lambda i: (0, i, 0)) if from_pre
              else pl.BlockSpec((blk, 8), lambda i: (i, 0)))
    return pl.pallas_call(
        body,
        grid=(grid,),
        in_specs=[
            e_spec,
            pl.BlockSpec((6, HID), lambda i: (0, 0)),
            pl.BlockSpec((32, 4), lambda i: (0, 0)),
            pl.BlockSpec((6, HID), lambda i: (0, 0)),
            pl.BlockSpec((32, 4), lambda i: (0, 0)),
            pl.BlockSpec((6, OUT), lambda i: (0, 0)),
            pl.BlockSpec((64, 1), lambda i: (0, 0)),
        ],
        out_specs=[
            pl.BlockSpec((blk, 16), lambda i: (i, 0)),
            pl.BlockSpec((blk, 16), lambda i: (i, 0)),
            pl.BlockSpec((blk, 16), lambda i: (i, 0)),
            pl.BlockSpec((8, 128), lambda i: (0, 0)),
        ],
        out_shape=[
            jax.ShapeDtypeStruct((rows, 16), jnp.float32),
            jax.ShapeDtypeStruct((rows, 16), jnp.float32),
            jax.ShapeDtypeStruct((rows, 16), jnp.float32),
            jax.ShapeDtypeStruct((8, 128), jnp.float32),
        ],
    )


def _fin_tc(heads, feat, width, with_bn):
    blk = 1000
    grid = N // blk

    def body(o_ref, b_ref, g_ref, be_ref, h_ref):
        ch = feat // heads
        acc = o_ref[0] + o_ref[1]
        den = acc[:, feat:feat + heads] + jnp.float32(1e-16)
        f_id = lax.broadcasted_iota(jnp.int32, (heads, feat), 1)
        h_id = lax.broadcasted_iota(jnp.int32, (heads, feat), 0)
        expand = jnp.where((f_id // ch) == h_id, 1.0, 0.0)
        den_e = jnp.dot(den, expand, preferred_element_type=jnp.float32)
        h = acc[:, :feat] / den_e + b_ref[...]
        if with_bn:
            h = h / jnp.sqrt(jnp.float32(1.0 + 1e-5)) * g_ref[...] + be_ref[...]
        h_ref[...] = jnp.where(h > 0, h, jnp.exp(h) - 1.0)

    return pl.pallas_call(
        body,
        grid=(grid,),
        in_specs=[
            pl.BlockSpec((2, blk, width), lambda i: (0, i, 0)),
            pl.BlockSpec((1, feat), lambda i: (0, 0)),
            pl.BlockSpec((1, feat), lambda i: (0, 0)),
            pl.BlockSpec((1, feat), lambda i: (0, 0)),
        ],
        out_specs=pl.BlockSpec((blk, feat), lambda i: (i, 0)),
        out_shape=jax.ShapeDtypeStruct((N, feat), jnp.float32),
    )


_fin_k1 = _fin_tc(4, HID, 144, True)
_fin_k3 = _fin_tc(1, OUT, 80, False)


def _pool_tc():
    blk = 1000
    grid = N // blk

    def body(h_ref, rum_ref, bat_ref, o_ref):
        i = pl.program_id(0)

        @pl.when(i == 0)
        def _():
            o_ref[...] = jnp.full((B, OUT), -jnp.inf, jnp.float32)

        hb = h_ref[...]
        seg = jnp.where(rum_ref[...] == 1, bat_ref[...], B)
        rid = lax.broadcasted_iota(jnp.int32, (B, 1), 0)
        acc = o_ref[...]
        for b in range(B):
            mb = jnp.max(jnp.where(seg == b, hb, -jnp.inf), axis=0,
                         keepdims=True)
            acc = jnp.where(rid == b, jnp.maximum(acc, mb), acc)
        o_ref[...] = acc

        @pl.when(i == grid - 1)
        def _():
            v = o_ref[...]
            o_ref[...] = jnp.where(jnp.isneginf(v), 0.0, v)

    return pl.pallas_call(
        body,
        grid=(grid,),
        in_specs=[
            pl.BlockSpec((blk, OUT), lambda i: (i, 0)),
            pl.BlockSpec((blk, 1), lambda i: (i, 0)),
            pl.BlockSpec((blk, 1), lambda i: (i, 0)),
        ],
        out_specs=pl.BlockSpec((B, OUT), lambda i: (0, 0)),
        out_shape=jax.ShapeDtypeStruct((B, OUT), jnp.float32),
    )


_pool_k = _pool_tc()

_Z144 = None


def _gat_layer(h_pad, s1, d1, q16, qmax, W, ast, adt, bias, g, be, heads,
               prep_fn, fin_fn):
    feat = W.shape[1]
    width = 144 if heads == 4 else 80
    xp_pad, adw, mx = prep_fn(h_pad, W, ast, adt)
    m = mx[0, :heads] + mx[1, :heads] + qmax
    m16 = jnp.zeros((16,), jnp.float32).at[:heads].set(m)
    zw = jnp.zeros((NP, width), jnp.float32)

    fn = _edge_kernel_12 if heads == 4 else _edge_kernel_3
    o = fn(s1, d1, q16, xp_pad, adw, m16, zw)
    return fin_fn(o, bias.reshape(1, feat), g.reshape(1, feat),
                  be.reshape(1, feat))


def kernel(x, edge_index, edge_attr, repeat_unit_mask, batch,
           W1, as1, ad1, We1, ae1, b1, g1, be1,
           W2, as2, ad2, We2, ae2, b2, g2, be2,
           W3, as3, ad3, We3, ae3, b3):
    src, dst = edge_index[0], edge_index[1]
    pre = _pre_pass(dst, edge_attr)

    sl = jnp.arange(N, dtype=jnp.int32)
    pad = PE - EN
    s1 = jnp.concatenate([src, sl, jnp.zeros((pad,), jnp.int32)])
    d1 = jnp.concatenate([dst, sl, (jnp.arange(pad, dtype=jnp.int32) % N)])

    eap8 = jnp.concatenate(
        [edge_attr, jnp.ones((E, 1), jnp.float32),
         jnp.zeros((E, 1), jnp.float32)], axis=1)
    EPAD = _pad_edges(E, 1280)
    eap8 = jnp.concatenate(
        [eap8, jnp.zeros((EPAD - E, 8), jnp.float32)], axis=0)
    qe1, qe2, qe3, mxe = _q_edges(eap8, We1, ae1.T, We2, ae2.T, We3, ae3.T)
    ql1, ql2, ql3, mxl = _q_loops(pre, We1, ae1.T, We2, ae2.T, We3, ae3.T)
    zq = jnp.zeros((PE - E - NP, 16), jnp.float32)
    q16_1 = jnp.concatenate([qe1[:E], ql1, zq], axis=0)
    q16_2 = jnp.concatenate([qe2[:E], ql2, zq], axis=0)
    q16_3 = jnp.concatenate([qe3[:E], ql3, zq], axis=0)
    qmax = jnp.maximum(mxe, mxl)
    qm1, qm2, qm3 = qmax[0, :4], qmax[1, :4], qmax[2, :1]

    xpad = jnp.concatenate([x, jnp.zeros((NP - N, 25), jnp.float32)], axis=0)
    h = _gat_layer(xpad, s1, d1, q16_1, qm1, W1, as1.T, ad1.T, b1, g1, be1,
                   4, _prep_k1, _fin_k1)
    hpad = jnp.concatenate([h, jnp.zeros((NP - N, HID), jnp.float32)], axis=0)
    h = _gat_layer(hpad, s1, d1, q16_2, qm2, W2, as2.T, ad2.T, b2, g2, be2,
                   4, _prep_k2, _fin_k1)
    hpad = jnp.concatenate([h, jnp.zeros((NP - N, HID), jnp.float32)], axis=0)
    h = _gat_layer(hpad, s1, d1, q16_3, qm3, W3, as3.T, ad3.T, b3, b3, b3,
                   1, _prep_k3, _fin_k3)

    out = _pool_k(h, repeat_unit_mask.reshape(N, 1), batch.reshape(N, 1))
    return out


_q_edges = _q_tc(_pad_edges(E, 1280), False)
_q_loops = _q_tc(NP, True)


# speedup vs baseline: 1.4883x; 1.1263x over previous
import functools

import jax
import jax.numpy as jnp
from jax import lax
from jax.experimental import pallas as pl
from jax.experimental.pallas import tpu as pltpu
from jax.experimental.pallas import tpu_sc as plsc

N = 10000
E = 640000
HID = 128
OUT = 64
B = 64

NC = 2
NS = 16
KH = 128
K = 8 * KH
KE = 96
NP = 10240
RPT = NP // NS

_mesh = functools.partial(
    plsc.VectorSubcoreMesh, core_axis_name="c", subcore_axis_name="s")


def _pad_edges(n, multiple):
    return ((n + multiple - 1) // multiple) * multiple


PE_PRE = _pad_edges(E, NC * NS * K)
EN = E + N
PE = _pad_edges(EN, NC * NS * KE)


def _pre_body(d1_hbm, eap_hbm, z8_hbm, out_hbm, d_v, e_v, acc):
    c = lax.axis_index("c")
    t = lax.axis_index("s")
    pltpu.sync_copy(z8_hbm.at[pl.ds(t * RPT, RPT)], acc.at[pl.ds(t * RPT, RPT)])
    plsc.subcore_barrier()
    epw = PE_PRE // (NC * NS)
    nwin = epw // K
    base_e = (c * NS + t) * epw

    def win(g, carry):
        e0 = base_e + g * K
        pltpu.sync_copy(eap_hbm.at[pl.ds(e0, K)], e_v)
        for j in range(K // KH):
            pltpu.sync_copy(d1_hbm.at[pl.ds(e0 + j * KH, KH)], d_v.at[j])
        for j in range(K // KH):
            pltpu.sync_copy(e_v.at[pl.ds(j * KH, KH)],
                            acc.at[d_v.at[j]], add=True)
        return carry

    lax.fori_loop(0, nwin, win, 0)
    plsc.subcore_barrier()
    pltpu.sync_copy(acc.at[pl.ds(t * RPT, RPT)],
                    out_hbm.at[c, pl.ds(t * RPT, RPT)])


def _pre_pass(dst, ea):
    pad = PE_PRE - E
    d_pad = jnp.concatenate([dst, (jnp.arange(pad, dtype=jnp.int32) % N)])
    eap = jnp.concatenate(
        [ea, jnp.ones((E, 1), jnp.float32), jnp.zeros((E, 1), jnp.float32)],
        axis=1)
    eap = jnp.concatenate([eap, jnp.zeros((pad, 8), jnp.float32)], axis=0)
    z8 = jnp.zeros((NP, 8), jnp.float32)

    fn = pl.kernel(
        _pre_body,
        out_type=jax.ShapeDtypeStruct((NC, NP, 8), jnp.float32),
        mesh=_mesh(),
        compiler_params=pltpu.CompilerParams(
            needs_layout_passes=False, use_tc_tiling_on_sc=False),
        scratch_types=[
            pltpu.VMEM((K // KH, KH), jnp.int32),
            pltpu.VMEM((K, 8), jnp.float32),
            pltpu.VMEM_SHARED((NP, 8), jnp.float32),
        ],
    )
    return fn(d_pad, eap, z8)


def _make_edge_kernel(heads, feat, width, loff):
    ch = feat // heads
    epw = PE // (NC * NS)
    nwin = epw // KE
    assert nwin % 2 == 0

    def body(s1_hbm, d1_hbm, q16_hbm, xp_hbm, adw_hbm, m16_hbm,
             zw_hbm, out_hbm,
             s_f, d_f, g_v, ad_v, q_v, m_v, acc, sg0, sg1, ss0, ss1):
        sem_g = (sg0, sg1)
        sem_s = (ss0, ss1)
        c = lax.axis_index("c")
        t = lax.axis_index("s")
        pltpu.sync_copy(m16_hbm, m_v)
        pltpu.sync_copy(zw_hbm.at[pl.ds(t * RPT, RPT)],
                        acc.at[pl.ds(t * RPT, RPT)])
        plsc.subcore_barrier()
        mvec = m_v[...]
        base_e = (c * NS + t) * epw
        lanes = lax.iota(jnp.int32, 16)
        lane_ok = jnp.logical_and(lanes >= loff, lanes < loff + heads)

        def loads_sq(w, b):
            pltpu.sync_copy(s1_hbm.at[pl.ds(base_e + w * KE, KE)], s_f.at[b])
            pltpu.sync_copy(q16_hbm.at[pl.ds(base_e + w * KE, KE)], q_v.at[b])

        def load_d(w, b):
            pltpu.sync_copy(d1_hbm.at[pl.ds(base_e + w * KE, KE)], d_f.at[b])

        def issue_gathers(b):
            pltpu.async_copy(xp_hbm.at[s_f.at[b]], g_v.at[b], sem_g[b])
            pltpu.async_copy(adw_hbm.at[d_f.at[b]], ad_v.at[b], sem_g[b])

        def drain_gathers(b):
            pltpu.make_async_copy(
                xp_hbm.at[s_f.at[b]], g_v.at[b], sem_g[b]).wait()
            pltpu.make_async_copy(
                adw_hbm.at[d_f.at[b]], ad_v.at[b], sem_g[b]).wait()

        def issue_scatter(b):
            pltpu.async_copy(g_v.at[b], acc.at[d_f.at[b]], sem_s[b], add=True)

        def drain_scatter(b):
            pltpu.make_async_copy(
                g_v.at[b], acc.at[d_f.at[b]], sem_s[b]).wait()

        def compute(w, b):
            e0 = base_e + w * KE

            def ebody(e, cy):
                a1 = g_v[b, e, pl.ds(feat, 16)]
                al = a1 + ad_v[b, e] + q_v[b, e]
                al = jnp.where(al > 0, al, al * jnp.float32(0.2))
                pe = jnp.exp(al - mvec)
                ok = jnp.logical_and(lane_ok, e0 + e < EN)
                pe = jnp.where(ok, pe, jnp.float32(0.0))
                g_v[b, e, pl.ds(feat, 16)] = pe
                for h in range(heads):
                    pb = jnp.broadcast_to(pe[loff + h], (16,))
                    for v2 in range(ch // 16):
                        col = h * ch + v2 * 16
                        g_v[b, e, pl.ds(col, 16)] = (
                            g_v[b, e, pl.ds(col, 16)] * pb)
                return cy

            lax.fori_loop(0, KE, ebody, 0, unroll=4)

        loads_sq(0, 0)
        load_d(0, 0)
        issue_gathers(0)
        loads_sq(1, 1)

        def win2(g2, carry):
            for b in (0, 1):
                w = g2 * 2 + b
                b2 = 1 - b
                drain_gathers(b)

                @pl.when(w > 0)
                def _():
                    drain_scatter(b2)

                @pl.when(w + 1 < nwin)
                def _():
                    load_d(w + 1, b2)
                    issue_gathers(b2)

                compute(w, b)
                issue_scatter(b)

                @pl.when(w + 2 < nwin)
                def _():
                    loads_sq(w + 2, b)
            return carry

        lax.fori_loop(0, nwin // 2, win2, 0)
        drain_scatter(1)
        plsc.subcore_barrier()
        pltpu.sync_copy(acc.at[pl.ds(t * RPT, RPT)],
                        out_hbm.at[c, pl.ds(t * RPT, RPT)])

    return pl.kernel(
        body,
        out_type=jax.ShapeDtypeStruct((NC, NP, width), jnp.float32),
        mesh=_mesh(),
        compiler_params=pltpu.CompilerParams(
            needs_layout_passes=False, use_tc_tiling_on_sc=False),
        scratch_types=[
            pltpu.VMEM((2, KE), jnp.int32),
            pltpu.VMEM((2, KE), jnp.int32),
            pltpu.VMEM((2, KE, width), jnp.float32),
            pltpu.VMEM((2, KE, 16), jnp.float32),
            pltpu.VMEM((2, KE, 16), jnp.float32),
            pltpu.VMEM((16,), jnp.float32),
            pltpu.VMEM_SHARED((NP, width), jnp.float32),
            pltpu.SemaphoreType.DMA,
            pltpu.SemaphoreType.DMA,
            pltpu.SemaphoreType.DMA,
            pltpu.SemaphoreType.DMA,
        ],
    )


_edge_kernel_1 = _make_edge_kernel(4, HID, 144, 0)
_edge_kernel_2 = _make_edge_kernel(4, HID, 144, 4)
_edge_kernel_3 = _make_edge_kernel(1, OUT, 80, 8)


_NEG = -3.0e38


def _prep_tc(heads, feat, width, fin, loff):
    blk = 1280
    grid = NP // blk

    def body(h_ref, w_ref, ast_ref, adt_ref, xp_ref, adw_ref, mx_ref):
        i = pl.program_id(0)
        ch = feat // heads
        xpb = jnp.dot(h_ref[...], w_ref[...],
                      preferred_element_type=jnp.float32)
        f_id = lax.broadcasted_iota(jnp.int32, (feat, heads), 0)
        h_id = lax.broadcasted_iota(jnp.int32, (feat, heads), 1)
        sel = (f_id // ch) == h_id
        amat_s = jnp.where(sel, jnp.tile(ast_ref[...], (heads, 1)), 0.0)
        amat_d = jnp.where(sel, jnp.tile(adt_ref[...], (heads, 1)), 0.0)
        asv = jnp.dot(xpb, amat_s, preferred_element_type=jnp.float32)
        adv = jnp.dot(xpb, amat_d, preferred_element_type=jnp.float32)
        xp_ref[:, :feat] = xpb
        xp_ref[:, feat:] = jnp.pad(
            asv, ((0, 0), (loff, width - feat - heads - loff)))
        adw_ref[...] = jnp.pad(adv, ((0, 0), (loff, 16 - heads - loff)))
        mrow_s = jnp.pad(jnp.max(asv, axis=0, keepdims=True),
                         ((0, 0), (0, 128 - heads)), constant_values=_NEG)
        mrow_d = jnp.pad(jnp.max(adv, axis=0, keepdims=True),
                         ((0, 0), (0, 128 - heads)), constant_values=_NEG)
        cur = jnp.pad(jnp.concatenate([mrow_s, mrow_d], axis=0),
                      ((0, 6), (0, 0)), constant_values=_NEG)

        @pl.when(i == 0)
        def _():
            mx_ref[...] = cur

        @pl.when(i > 0)
        def _():
            mx_ref[...] = jnp.maximum(mx_ref[...], cur)

    return pl.pallas_call(
        body,
        grid=(grid,),
        in_specs=[
            pl.BlockSpec((blk, fin), lambda i: (i, 0)),
            pl.BlockSpec((fin, feat), lambda i: (0, 0)),
            pl.BlockSpec((feat // heads, heads), lambda i: (0, 0)),
            pl.BlockSpec((feat // heads, heads), lambda i: (0, 0)),
        ],
        out_specs=[
            pl.BlockSpec((blk, width), lambda i: (i, 0)),
            pl.BlockSpec((blk, 16), lambda i: (i, 0)),
            pl.BlockSpec((8, 128), lambda i: (0, 0)),
        ],
        out_shape=[
            jax.ShapeDtypeStruct((NP, width), jnp.float32),
            jax.ShapeDtypeStruct((NP, 16), jnp.float32),
            jax.ShapeDtypeStruct((8, 128), jnp.float32),
        ],
    )


_prep_k1 = _prep_tc(4, HID, 144, 25, 0)
_prep_k2 = _prep_tc(4, HID, 144, HID, 4)
_prep_k3 = _prep_tc(1, OUT, 80, HID, 8)


def _q_tc(rows, from_pre):
    blk = 1280
    grid = rows // blk

    def body(e_ref, we1_ref, ae1_ref, we2_ref, ae2_ref, we3_ref, ae3_ref,
             q_ref, mx_ref):
        i = pl.program_id(0)
        if from_pre:
            s = e_ref[0] + e_ref[1]
            deg = jnp.maximum(s[:, 6:7], 1.0)
            ea = s[:, :6] / deg
        else:
            ea = e_ref[:, :6]

        def fold(we_ref, ae_ref, heads, ch):
            w3 = we_ref[...]
            at = ae_ref[...]
            f_id = lax.broadcasted_iota(jnp.int32, (heads * ch, heads), 0)
            h_id = lax.broadcasted_iota(jnp.int32, (heads * ch, heads), 1)
            sel = (f_id // ch) == h_id
            amat = jnp.where(sel, jnp.tile(at, (heads, 1)), 0.0)
            return jnp.dot(w3, amat, preferred_element_type=jnp.float32)

        ae_mats = [fold(we1_ref, ae1_ref, 4, 32),
                   fold(we2_ref, ae2_ref, 4, 32),
                   fold(we3_ref, ae3_ref, 1, 64)]
        rows_m = []
        qs = []
        for (am, heads) in ((ae_mats[0], 4), (ae_mats[1], 4),
                            (ae_mats[2], 1)):
            q = jnp.dot(ea, am, preferred_element_type=jnp.float32)
            qs.append(q)
            rows_m.append(jnp.pad(jnp.max(q, axis=0, keepdims=True),
                                  ((0, 0), (0, 128 - heads)),
                                  constant_values=_NEG))
        q_ref[...] = jnp.pad(jnp.concatenate(qs, axis=1),
                             ((0, 0), (0, 7)))
        cur = jnp.pad(jnp.concatenate(rows_m, axis=0), ((0, 5), (0, 0)),
                      constant_values=_NEG)

        @pl.when(i == 0)
        def _():
            mx_ref[...] = cur

        @pl.when(i > 0)
        def _():
            mx_ref[...] = jnp.maximum(mx_ref[...], cur)

    e_spec = (pl.BlockSpec((2, blk, 8), lambda i: (0, i, 0)) if from_pre
              else pl.BlockSpec((blk, 8), lambda i: (i, 0)))
    return pl.pallas_call(
        body,
        grid=(grid,),
        in_specs=[
            e_spec,
            pl.BlockSpec((6, HID), lambda i: (0, 0)),
            pl.BlockSpec((32, 4), lambda i: (0, 0)),
            pl.BlockSpec((6, HID), lambda i: (0, 0)),
            pl.BlockSpec((32, 4), lambda i: (0, 0)),
            pl.BlockSpec((6, OUT), lambda i: (0, 0)),
            pl.BlockSpec((64, 1), lambda i: (0, 0)),
        ],
        out_specs=[
            pl.BlockSpec((blk, 16), lambda i: (i, 0)),
            pl.BlockSpec((8, 128), lambda i: (0, 0)),
        ],
        out_shape=[
            jax.ShapeDtypeStruct((rows, 16), jnp.float32),
            jax.ShapeDtypeStruct((8, 128), jnp.float32),
        ],
    )


def _fin_tc(heads, feat, width, with_bn, loff):
    blk = 1000
    grid = N // blk

    def body(o_ref, b_ref, g_ref, be_ref, h_ref):
        ch = feat // heads
        acc = o_ref[0] + o_ref[1]
        den = acc[:, feat + loff:feat + loff + heads] + jnp.float32(1e-16)
        f_id = lax.broadcasted_iota(jnp.int32, (heads, feat), 1)
        h_id = lax.broadcasted_iota(jnp.int32, (heads, feat), 0)
        expand = jnp.where((f_id // ch) == h_id, 1.0, 0.0)
        den_e = jnp.dot(den, expand, preferred_element_type=jnp.float32)
        h = acc[:, :feat] / den_e + b_ref[...]
        if with_bn:
            h = h / jnp.sqrt(jnp.float32(1.0 + 1e-5)) * g_ref[...] + be_ref[...]
        h_ref[...] = jnp.where(h > 0, h, jnp.exp(h) - 1.0)

    return pl.pallas_call(
        body,
        grid=(grid,),
        in_specs=[
            pl.BlockSpec((2, blk, width), lambda i: (0, i, 0)),
            pl.BlockSpec((1, feat), lambda i: (0, 0)),
            pl.BlockSpec((1, feat), lambda i: (0, 0)),
            pl.BlockSpec((1, feat), lambda i: (0, 0)),
        ],
        out_specs=pl.BlockSpec((blk, feat), lambda i: (i, 0)),
        out_shape=jax.ShapeDtypeStruct((N, feat), jnp.float32),
    )


_fin_k1 = _fin_tc(4, HID, 144, True, 0)
_fin_k2 = _fin_tc(4, HID, 144, True, 4)
_fin_k3 = _fin_tc(1, OUT, 80, False, 8)


def _pool_tc():
    blk = 1000
    grid = N // blk

    def body(h_ref, rum_ref, bat_ref, o_ref):
        i = pl.program_id(0)

        @pl.when(i == 0)
        def _():
            o_ref[...] = jnp.full((B, OUT), -jnp.inf, jnp.float32)

        hb = h_ref[...]
        seg = jnp.where(rum_ref[...] == 1, bat_ref[...], B)
        rid = lax.broadcasted_iota(jnp.int32, (B, 1), 0)
        acc = o_ref[...]
        for b in range(B):
            mb = jnp.max(jnp.where(seg == b, hb, -jnp.inf), axis=0,
                         keepdims=True)
            acc = jnp.where(rid == b, jnp.maximum(acc, mb), acc)
        o_ref[...] = acc

        @pl.when(i == grid - 1)
        def _():
            v = o_ref[...]
            o_ref[...] = jnp.where(jnp.isneginf(v), 0.0, v)

    return pl.pallas_call(
        body,
        grid=(grid,),
        in_specs=[
            pl.BlockSpec((blk, OUT), lambda i: (i, 0)),
            pl.BlockSpec((blk, 1), lambda i: (i, 0)),
            pl.BlockSpec((blk, 1), lambda i: (i, 0)),
        ],
        out_specs=pl.BlockSpec((B, OUT), lambda i: (0, 0)),
        out_shape=jax.ShapeDtypeStruct((B, OUT), jnp.float32),
    )


_pool_k = _pool_tc()

_Z144 = None


def _gat_layer(h_pad, s1, d1, q16, qmax, W, ast, adt, bias, g, be, heads,
               loff, prep_fn, edge_fn, fin_fn):
    feat = W.shape[1]
    width = 144 if heads == 4 else 80
    xp_pad, adw, mx = prep_fn(h_pad, W, ast, adt)
    m = mx[0, :heads] + mx[1, :heads] + qmax
    m16 = jnp.zeros((16,), jnp.float32).at[loff:loff + heads].set(m)
    zw = jnp.zeros((NP, width), jnp.float32)

    o = edge_fn(s1, d1, q16, xp_pad, adw, m16, zw)
    return fin_fn(o, bias.reshape(1, feat), g.reshape(1, feat),
                  be.reshape(1, feat))


def kernel(x, edge_index, edge_attr, repeat_unit_mask, batch,
           W1, as1, ad1, We1, ae1, b1, g1, be1,
           W2, as2, ad2, We2, ae2, b2, g2, be2,
           W3, as3, ad3, We3, ae3, b3):
    src, dst = edge_index[0], edge_index[1]
    pre = _pre_pass(dst, edge_attr)

    sl = jnp.arange(N, dtype=jnp.int32)
    pad = PE - EN
    s1 = jnp.concatenate([src, sl, jnp.zeros((pad,), jnp.int32)])
    d1 = jnp.concatenate([dst, sl, (jnp.arange(pad, dtype=jnp.int32) % N)])

    eap8 = jnp.concatenate(
        [edge_attr, jnp.ones((E, 1), jnp.float32),
         jnp.zeros((E, 1), jnp.float32)], axis=1)
    EPAD = _pad_edges(E, 1280)
    eap8 = jnp.concatenate(
        [eap8, jnp.zeros((EPAD - E, 8), jnp.float32)], axis=0)
    qe, mxe = _q_edges(eap8, We1, ae1.T, We2, ae2.T, We3, ae3.T)
    ql, mxl = _q_loops(pre, We1, ae1.T, We2, ae2.T, We3, ae3.T)
    zq = jnp.zeros((PE - E - NP, 16), jnp.float32)
    q16 = jnp.concatenate([qe[:E], ql, zq], axis=0)
    qmax = jnp.maximum(mxe, mxl)
    qm1, qm2, qm3 = qmax[0, :4], qmax[1, :4], qmax[2, :1]

    xpad = jnp.concatenate([x, jnp.zeros((NP - N, 25), jnp.float32)], axis=0)
    h = _gat_layer(xpad, s1, d1, q16, qm1, W1, as1.T, ad1.T, b1, g1, be1,
                   4, 0, _prep_k1, _edge_kernel_1, _fin_k1)
    hpad = jnp.concatenate([h, jnp.zeros((NP - N, HID), jnp.float32)], axis=0)
    h = _gat_layer(hpad, s1, d1, q16, qm2, W2, as2.T, ad2.T, b2, g2, be2,
                   4, 4, _prep_k2, _edge_kernel_2, _fin_k2)
    hpad = jnp.concatenate([h, jnp.zeros((NP - N, HID), jnp.float32)], axis=0)
    h = _gat_layer(hpad, s1, d1, q16, qm3, W3, as3.T, ad3.T, b3, b3, b3,
                   1, 8, _prep_k3, _edge_kernel_3, _fin_k3)

    out = _pool_k(h, repeat_unit_mask.reshape(N, 1), batch.reshape(N, 1))
    return out


_q_edges = _q_tc(_pad_edges(E, 1280), False)
_q_loops = _q_tc(NP, True)


# speedup vs baseline: 1.5739x; 1.0575x over previous
import functools

import jax
import jax.numpy as jnp
from jax import lax
from jax.experimental import pallas as pl
from jax.experimental.pallas import tpu as pltpu
from jax.experimental.pallas import tpu_sc as plsc

N = 10000
E = 640000
HID = 128
OUT = 64
B = 64

NC = 2
NS = 16
KH = 128
K = 8 * KH
KE = 96
NP = 10240
RPT = NP // NS

_mesh = functools.partial(
    plsc.VectorSubcoreMesh, core_axis_name="c", subcore_axis_name="s")


def _pad_edges(n, multiple):
    return ((n + multiple - 1) // multiple) * multiple


PE_PRE = _pad_edges(E, NC * NS * K)
EN = E + N
PE = _pad_edges(EN, NC * NS * KE)


def _pre_body(d1_hbm, eap_hbm, z8_hbm, out_hbm, d_v, e_v, acc):
    c = lax.axis_index("c")
    t = lax.axis_index("s")
    pltpu.sync_copy(z8_hbm.at[pl.ds(t * RPT, RPT)], acc.at[pl.ds(t * RPT, RPT)])
    plsc.subcore_barrier()
    epw = PE_PRE // (NC * NS)
    nwin = epw // K
    base_e = (c * NS + t) * epw

    def win(g, carry):
        e0 = base_e + g * K
        pltpu.sync_copy(eap_hbm.at[pl.ds(e0, K)], e_v)
        for j in range(K // KH):
            pltpu.sync_copy(d1_hbm.at[pl.ds(e0 + j * KH, KH)], d_v.at[j])
        for j in range(K // KH):
            pltpu.sync_copy(e_v.at[pl.ds(j * KH, KH)],
                            acc.at[d_v.at[j]], add=True)
        return carry

    lax.fori_loop(0, nwin, win, 0)
    plsc.subcore_barrier()
    pltpu.sync_copy(acc.at[pl.ds(t * RPT, RPT)],
                    out_hbm.at[c, pl.ds(t * RPT, RPT)])


def _pre_pass(dst, ea):
    pad = PE_PRE - E
    d_pad = jnp.concatenate([dst, (jnp.arange(pad, dtype=jnp.int32) % N)])
    eap = jnp.concatenate(
        [ea, jnp.ones((E, 1), jnp.float32), jnp.zeros((E, 1), jnp.float32)],
        axis=1)
    eap = jnp.concatenate([eap, jnp.zeros((pad, 8), jnp.float32)], axis=0)
    z8 = jnp.zeros((NP, 8), jnp.float32)

    fn = pl.kernel(
        _pre_body,
        out_type=jax.ShapeDtypeStruct((NC, NP, 8), jnp.float32),
        mesh=_mesh(),
        compiler_params=pltpu.CompilerParams(
            needs_layout_passes=False, use_tc_tiling_on_sc=False),
        scratch_types=[
            pltpu.VMEM((K // KH, KH), jnp.int32),
            pltpu.VMEM((K, 8), jnp.float32),
            pltpu.VMEM_SHARED((NP, 8), jnp.float32),
        ],
    )
    return fn(d_pad, eap, z8)


def _make_edge_kernel(heads, feat, width, loff):
    ch = feat // heads
    epw = PE // (NC * NS)
    nwin = epw // KE
    assert nwin % 2 == 0

    def body(s1_hbm, d1_hbm, q16_hbm, xp_hbm, adw_hbm, m16_hbm,
             zw_hbm, out_hbm,
             s_f, d_f, g_v, ad_v, q_v, m_v, acc, sg0, sg1, ss0, ss1):
        sem_g = (sg0, sg1)
        sem_s = (ss0, ss1)
        c = lax.axis_index("c")
        t = lax.axis_index("s")
        pltpu.sync_copy(m16_hbm, m_v)
        pltpu.sync_copy(zw_hbm.at[pl.ds(t * RPT, RPT)],
                        acc.at[pl.ds(t * RPT, RPT)])
        plsc.subcore_barrier()
        mvec = m_v[...]
        base_e = (c * NS + t) * epw
        lanes = lax.iota(jnp.int32, 16)
        lane_ok = jnp.logical_and(lanes >= loff, lanes < loff + heads)

        def loads_sq(w, b):
            pltpu.sync_copy(s1_hbm.at[pl.ds(base_e + w * KE, KE)], s_f.at[b])
            pltpu.sync_copy(q16_hbm.at[pl.ds(base_e + w * KE, KE)], q_v.at[b])

        def load_d(w, b):
            pltpu.sync_copy(d1_hbm.at[pl.ds(base_e + w * KE, KE)], d_f.at[b])

        def issue_gathers(b):
            pltpu.async_copy(xp_hbm.at[s_f.at[b]], g_v.at[b], sem_g[b])
            pltpu.async_copy(adw_hbm.at[d_f.at[b]], ad_v.at[b], sem_g[b])

        def drain_gathers(b):
            pltpu.make_async_copy(
                xp_hbm.at[s_f.at[b]], g_v.at[b], sem_g[b]).wait()
            pltpu.make_async_copy(
                adw_hbm.at[d_f.at[b]], ad_v.at[b], sem_g[b]).wait()

        def issue_scatter(b):
            pltpu.async_copy(g_v.at[b], acc.at[d_f.at[b]], sem_s[b], add=True)

        def drain_scatter(b):
            pltpu.make_async_copy(
                g_v.at[b], acc.at[d_f.at[b]], sem_s[b]).wait()

        def compute(w, b):
            e0 = base_e + w * KE

            def ebody(e, cy):
                a1 = g_v[b, e, pl.ds(feat, 16)]
                al = a1 + ad_v[b, e] + q_v[b, e]
                al = jnp.where(al > 0, al, al * jnp.float32(0.2))
                pe = jnp.exp(al - mvec)
                ok = jnp.logical_and(lane_ok, e0 + e < EN)
                pe = jnp.where(ok, pe, jnp.float32(0.0))
                g_v[b, e, pl.ds(feat, 16)] = pe
                for h in range(heads):
                    pb = jnp.broadcast_to(pe[loff + h], (16,))
                    for v2 in range(ch // 16):
                        col = h * ch + v2 * 16
                        g_v[b, e, pl.ds(col, 16)] = (
                            g_v[b, e, pl.ds(col, 16)] * pb)
                return cy

            lax.fori_loop(0, KE, ebody, 0, unroll=4)

        loads_sq(0, 0)
        load_d(0, 0)
        issue_gathers(0)
        loads_sq(1, 1)

        def win2(g2, carry):
            for b in (0, 1):
                w = g2 * 2 + b
                b2 = 1 - b
                drain_gathers(b)

                @pl.when(w > 0)
                def _():
                    drain_scatter(b2)

                @pl.when(w + 1 < nwin)
                def _():
                    load_d(w + 1, b2)
                    issue_gathers(b2)

                compute(w, b)
                issue_scatter(b)

                @pl.when(w + 2 < nwin)
                def _():
                    loads_sq(w + 2, b)
            return carry

        lax.fori_loop(0, nwin // 2, win2, 0)
        drain_scatter(1)
        plsc.subcore_barrier()
        pltpu.sync_copy(acc.at[pl.ds(t * RPT, RPT)],
                        out_hbm.at[c, pl.ds(t * RPT, RPT)])

    return pl.kernel(
        body,
        out_type=jax.ShapeDtypeStruct((NC, NP, width), jnp.float32),
        mesh=_mesh(),
        compiler_params=pltpu.CompilerParams(
            needs_layout_passes=False, use_tc_tiling_on_sc=False),
        scratch_types=[
            pltpu.VMEM((2, KE), jnp.int32),
            pltpu.VMEM((2, KE), jnp.int32),
            pltpu.VMEM((2, KE, width), jnp.float32),
            pltpu.VMEM((2, KE, 16), jnp.float32),
            pltpu.VMEM((2, KE, 16), jnp.float32),
            pltpu.VMEM((16,), jnp.float32),
            pltpu.VMEM_SHARED((NP, width), jnp.float32),
            pltpu.SemaphoreType.DMA,
            pltpu.SemaphoreType.DMA,
            pltpu.SemaphoreType.DMA,
            pltpu.SemaphoreType.DMA,
        ],
    )


_edge_kernel_1 = _make_edge_kernel(4, HID, 144, 0)
_edge_kernel_2 = _make_edge_kernel(4, HID, 144, 4)
_edge_kernel_3 = _make_edge_kernel(1, OUT, 80, 8)


_NEG = -3.0e38


def _prep_tc(heads, feat, width, fin, loff):
    blk = 1280
    grid = NP // blk

    def body(h_ref, w_ref, ast_ref, adt_ref, xp_ref, adw_ref, mx_ref):
        i = pl.program_id(0)
        ch = feat // heads
        xpb = jnp.dot(h_ref[...], w_ref[...],
                      preferred_element_type=jnp.float32)
        f_id = lax.broadcasted_iota(jnp.int32, (feat, heads), 0)
        h_id = lax.broadcasted_iota(jnp.int32, (feat, heads), 1)
        sel = (f_id // ch) == h_id
        amat_s = jnp.where(sel, jnp.tile(ast_ref[...], (heads, 1)), 0.0)
        amat_d = jnp.where(sel, jnp.tile(adt_ref[...], (heads, 1)), 0.0)
        asv = jnp.dot(xpb, amat_s, preferred_element_type=jnp.float32)
        adv = jnp.dot(xpb, amat_d, preferred_element_type=jnp.float32)
        xp_ref[:, :feat] = xpb
        xp_ref[:, feat:] = jnp.pad(
            asv, ((0, 0), (loff, width - feat - heads - loff)))
        adw_ref[...] = jnp.pad(adv, ((0, 0), (loff, 16 - heads - loff)))
        mrow_s = jnp.pad(jnp.max(asv, axis=0, keepdims=True),
                         ((0, 0), (0, 128 - heads)), constant_values=_NEG)
        mrow_d = jnp.pad(jnp.max(adv, axis=0, keepdims=True),
                         ((0, 0), (0, 128 - heads)), constant_values=_NEG)
        cur = jnp.pad(jnp.concatenate([mrow_s, mrow_d], axis=0),
                      ((0, 6), (0, 0)), constant_values=_NEG)

        @pl.when(i == 0)
        def _():
            mx_ref[...] = cur

        @pl.when(i > 0)
        def _():
            mx_ref[...] = jnp.maximum(mx_ref[...], cur)

    return pl.pallas_call(
        body,
        grid=(grid,),
        in_specs=[
            pl.BlockSpec((blk, fin), lambda i: (i, 0)),
            pl.BlockSpec((fin, feat), lambda i: (0, 0)),
            pl.BlockSpec((feat // heads, heads), lambda i: (0, 0)),
            pl.BlockSpec((feat // heads, heads), lambda i: (0, 0)),
        ],
        out_specs=[
            pl.BlockSpec((blk, width), lambda i: (i, 0)),
            pl.BlockSpec((blk, 16), lambda i: (i, 0)),
            pl.BlockSpec((8, 128), lambda i: (0, 0)),
        ],
        out_shape=[
            jax.ShapeDtypeStruct((NP, width), jnp.float32),
            jax.ShapeDtypeStruct((NP, 16), jnp.float32),
            jax.ShapeDtypeStruct((8, 128), jnp.float32),
        ],
    )


_prep_k1 = _prep_tc(4, HID, 144, 25, 0)
_prep_k2 = _prep_tc(4, HID, 144, HID, 4)
_prep_k3 = _prep_tc(1, OUT, 80, HID, 8)


def _q_tc(rows, from_pre):
    blk = 2560
    grid = rows // blk

    def body(e_ref, we1_ref, ae1_ref, we2_ref, ae2_ref, we3_ref, ae3_ref,
             q_ref, mx_ref):
        i = pl.program_id(0)
        if from_pre:
            s = e_ref[0] + e_ref[1]
            deg = jnp.maximum(s[:, 6:7], 1.0)
            ea8 = s / deg
        else:
            ea8 = e_ref[...]

        def fold(we_ref, ae_ref, heads, ch):
            w3 = we_ref[...]
            at = ae_ref[...]
            f_id = lax.broadcasted_iota(jnp.int32, (heads * ch, heads), 0)
            h_id = lax.broadcasted_iota(jnp.int32, (heads * ch, heads), 1)
            sel = (f_id // ch) == h_id
            amat = jnp.where(sel, jnp.tile(at, (heads, 1)), 0.0)
            return jnp.dot(w3, amat, preferred_element_type=jnp.float32)

        acat = jnp.concatenate(
            [fold(we1_ref, ae1_ref, 4, 32), fold(we2_ref, ae2_ref, 4, 32),
             fold(we3_ref, ae3_ref, 1, 64), jnp.zeros((6, 7), jnp.float32)],
            axis=1)
        acat = jnp.pad(acat, ((0, 2), (0, 0)))
        q16 = jnp.dot(ea8, acat, preferred_element_type=jnp.float32)
        q_ref[...] = q16
        cur = jnp.pad(jnp.max(q16, axis=0, keepdims=True),
                      ((0, 7), (0, 112)), constant_values=_NEG)

        @pl.when(i == 0)
        def _():
            mx_ref[...] = cur

        @pl.when(i > 0)
        def _():
            mx_ref[...] = jnp.maximum(mx_ref[...], cur)

    e_spec = (pl.BlockSpec((2, blk, 8), lambda i: (0, i, 0)) if from_pre
              else pl.BlockSpec((blk, 8), lambda i: (i, 0)))
    return pl.pallas_call(
        body,
        grid=(grid,),
        in_specs=[
            e_spec,
            pl.BlockSpec((6, HID), lambda i: (0, 0)),
            pl.BlockSpec((32, 4), lambda i: (0, 0)),
            pl.BlockSpec((6, HID), lambda i: (0, 0)),
            pl.BlockSpec((32, 4), lambda i: (0, 0)),
            pl.BlockSpec((6, OUT), lambda i: (0, 0)),
            pl.BlockSpec((64, 1), lambda i: (0, 0)),
        ],
        out_specs=[
            pl.BlockSpec((blk, 16), lambda i: (i, 0)),
            pl.BlockSpec((8, 128), lambda i: (0, 0)),
        ],
        out_shape=[
            jax.ShapeDtypeStruct((rows, 16), jnp.float32),
            jax.ShapeDtypeStruct((8, 128), jnp.float32),
        ],
    )


def _fin_tc(heads, feat, width, with_bn, loff):
    blk = 1000
    grid = N // blk

    def body(o_ref, b_ref, g_ref, be_ref, h_ref):
        ch = feat // heads
        acc = o_ref[0] + o_ref[1]
        den = acc[:, feat + loff:feat + loff + heads] + jnp.float32(1e-16)
        f_id = lax.broadcasted_iota(jnp.int32, (heads, feat), 1)
        h_id = lax.broadcasted_iota(jnp.int32, (heads, feat), 0)
        expand = jnp.where((f_id // ch) == h_id, 1.0, 0.0)
        den_e = jnp.dot(den, expand, preferred_element_type=jnp.float32)
        h = acc[:, :feat] / den_e + b_ref[...]
        if with_bn:
            h = h / jnp.sqrt(jnp.float32(1.0 + 1e-5)) * g_ref[...] + be_ref[...]
        h_ref[...] = jnp.where(h > 0, h, jnp.exp(h) - 1.0)

    return pl.pallas_call(
        body,
        grid=(grid,),
        in_specs=[
            pl.BlockSpec((2, blk, width), lambda i: (0, i, 0)),
            pl.BlockSpec((1, feat), lambda i: (0, 0)),
            pl.BlockSpec((1, feat), lambda i: (0, 0)),
            pl.BlockSpec((1, feat), lambda i: (0, 0)),
        ],
        out_specs=pl.BlockSpec((blk, feat), lambda i: (i, 0)),
        out_shape=jax.ShapeDtypeStruct((N, feat), jnp.float32),
    )


_fin_k1 = _fin_tc(4, HID, 144, True, 0)
_fin_k2 = _fin_tc(4, HID, 144, True, 4)
_fin_k3 = _fin_tc(1, OUT, 80, False, 8)


def _pool_tc():
    blk = 1000
    grid = N // blk

    def body(h_ref, rum_ref, bat_ref, o_ref):
        i = pl.program_id(0)

        @pl.when(i == 0)
        def _():
            o_ref[...] = jnp.full((B, OUT), -jnp.inf, jnp.float32)

        hb = h_ref[...]
        seg = jnp.where(rum_ref[...] == 1, bat_ref[...], B)
        rid = lax.broadcasted_iota(jnp.int32, (B, 1), 0)
        acc = o_ref[...]
        for b in range(B):
            mb = jnp.max(jnp.where(seg == b, hb, -jnp.inf), axis=0,
                         keepdims=True)
            acc = jnp.where(rid == b, jnp.maximum(acc, mb), acc)
        o_ref[...] = acc

        @pl.when(i == grid - 1)
        def _():
            v = o_ref[...]
            o_ref[...] = jnp.where(jnp.isneginf(v), 0.0, v)

    return pl.pallas_call(
        body,
        grid=(grid,),
        in_specs=[
            pl.BlockSpec((blk, OUT), lambda i: (i, 0)),
            pl.BlockSpec((blk, 1), lambda i: (i, 0)),
            pl.BlockSpec((blk, 1), lambda i: (i, 0)),
        ],
        out_specs=pl.BlockSpec((B, OUT), lambda i: (0, 0)),
        out_shape=jax.ShapeDtypeStruct((B, OUT), jnp.float32),
    )


_pool_k = _pool_tc()

_Z144 = None


def _gat_layer(h_pad, s1, d1, q16, qmax, W, ast, adt, bias, g, be, heads,
               loff, prep_fn, edge_fn, fin_fn):
    feat = W.shape[1]
    width = 144 if heads == 4 else 80
    xp_pad, adw, mx = prep_fn(h_pad, W, ast, adt)
    m = mx[0, :heads] + mx[1, :heads] + qmax
    m16 = jnp.zeros((16,), jnp.float32).at[loff:loff + heads].set(m)
    zw = jnp.zeros((NP, width), jnp.float32)

    o = edge_fn(s1, d1, q16, xp_pad, adw, m16, zw)
    return fin_fn(o, bias.reshape(1, feat), g.reshape(1, feat),
                  be.reshape(1, feat))


def kernel(x, edge_index, edge_attr, repeat_unit_mask, batch,
           W1, as1, ad1, We1, ae1, b1, g1, be1,
           W2, as2, ad2, We2, ae2, b2, g2, be2,
           W3, as3, ad3, We3, ae3, b3):
    src, dst = edge_index[0], edge_index[1]
    pre = _pre_pass(dst, edge_attr)

    sl = jnp.arange(N, dtype=jnp.int32)
    pad = PE - EN
    s1 = jnp.concatenate([src, sl, jnp.zeros((pad,), jnp.int32)])
    d1 = jnp.concatenate([dst, sl, (jnp.arange(pad, dtype=jnp.int32) % N)])

    eap8 = jnp.concatenate(
        [edge_attr, jnp.ones((E, 1), jnp.float32),
         jnp.zeros((E, 1), jnp.float32)], axis=1)
    EPAD = _pad_edges(E, 2560)
    eap8 = jnp.concatenate(
        [eap8, jnp.zeros((EPAD - E, 8), jnp.float32)], axis=0)
    qe, mxe = _q_edges(eap8, We1, ae1.T, We2, ae2.T, We3, ae3.T)
    ql, mxl = _q_loops(pre, We1, ae1.T, We2, ae2.T, We3, ae3.T)
    zq = jnp.zeros((PE - E - NP, 16), jnp.float32)
    q16 = jnp.concatenate([qe[:E], ql, zq], axis=0)
    qmax = jnp.maximum(mxe, mxl)
    qm1, qm2, qm3 = qmax[0, 0:4], qmax[0, 4:8], qmax[0, 8:9]

    xpad = jnp.concatenate([x, jnp.zeros((NP - N, 25), jnp.float32)], axis=0)
    h = _gat_layer(xpad, s1, d1, q16, qm1, W1, as1.T, ad1.T, b1, g1, be1,
                   4, 0, _prep_k1, _edge_kernel_1, _fin_k1)
    hpad = jnp.concatenate([h, jnp.zeros((NP - N, HID), jnp.float32)], axis=0)
    h = _gat_layer(hpad, s1, d1, q16, qm2, W2, as2.T, ad2.T, b2, g2, be2,
                   4, 4, _prep_k2, _edge_kernel_2, _fin_k2)
    hpad = jnp.concatenate([h, jnp.zeros((NP - N, HID), jnp.float32)], axis=0)
    h = _gat_layer(hpad, s1, d1, q16, qm3, W3, as3.T, ad3.T, b3, b3, b3,
                   1, 8, _prep_k3, _edge_kernel_3, _fin_k3)

    out = _pool_k(h, repeat_unit_mask.reshape(N, 1), batch.reshape(N, 1))
    return out


_q_edges = _q_tc(_pad_edges(E, 2560), False)
_q_loops = _q_tc(NP, True)
